# Initial kernel scaffold; baseline (speedup 1.0000x reference)
import jax
import jax.numpy as jnp
from jax.experimental import pallas as pl


def kernel(x, batch):
    raise NotImplementedError("write your pallas kernel here")


# speedup vs baseline: 4.1450x; 4.1450x over previous
import functools

import jax
import jax.numpy as jnp
from jax import lax
from jax.experimental import pallas as pl
from jax.experimental.pallas import tpu as pltpu
from jax.experimental.pallas import tpu_sc as plsc

NROWS = 100000
D = 512
NSEG = 512
L = 16
NC = 2
NS = 16
B = 2000
NBLK = NROWS // B


def _sc_body(x_hbm, ids_hbm, out_hbm,
             xbuf0, xbuf1, ibuf0, ibuf1, sum_out, max_out, sacc, macc,
             xsem0, xsem1, isem0, isem1):
    c = lax.axis_index("c")
    s = lax.axis_index("s")
    wid = s * NC + c
    c0 = wid * L

    zeros = jnp.zeros((L,), jnp.float32)
    ninf = jnp.full((L,), -jnp.inf, jnp.float32)

    def init_body(i, carry):
        sum_out[i] = zeros
        max_out[i] = ninf
        return carry
    lax.fori_loop(0, NSEG, init_body, 0)

    def start(g, xb, ib, xsem, isem):
        pltpu.async_copy(x_hbm.at[pl.ds(g * B, B), pl.ds(c0, L)], xb, xsem)
        pltpu.async_copy(ids_hbm.at[pl.ds(g * B, B)], ib, isem)

    def wait(g, xb, ib, xsem, isem):
        pltpu.make_async_copy(
            x_hbm.at[pl.ds(g * B, B), pl.ds(c0, L)], xb, xsem).wait()
        pltpu.make_async_copy(
            ids_hbm.at[pl.ds(g * B, B)], ib, isem).wait()

    def _tree(op, xs):
        while len(xs) > 1:
            xs = [op(xs[i], xs[i + 1]) for i in range(0, len(xs), 2)]
        return xs[0]

    def process(xb, ib, prev0):
        def group_body(k, prev):
            base = k * L
            idvec = ib[pl.ds(base, L)]
            vs = [xb[base + j] for j in range(L)]
            same = jnp.logical_and(idvec[0] == idvec[L - 1],
                                   idvec[0] == prev)

            @pl.when(same)
            def fast():
                sacc[...] = sacc[...] + _tree(lambda a, b: a + b, vs)
                macc[...] = jnp.maximum(macc[...], _tree(jnp.maximum, vs))

            @pl.when(jnp.logical_not(same))
            def slow():
                sa = sacc[...]
                ma = macc[...]
                p = prev
                for j in range(L):
                    sid = idvec[j]
                    new = sid != p

                    @pl.when(jnp.logical_and(new, p >= 0))
                    def _(sa=sa, ma=ma, p=p):
                        sum_out[p] = sa
                        max_out[p] = ma

                    sa = jnp.where(new, vs[j], sa + vs[j])
                    ma = jnp.where(new, vs[j], jnp.maximum(ma, vs[j]))
                    p = sid
                sacc[...] = sa
                macc[...] = ma

            return idvec[L - 1]
        return lax.fori_loop(0, B // L, group_body, prev0)

    start(0, xbuf0, ibuf0, xsem0, isem0)
    sacc[...] = zeros
    macc[...] = ninf

    def pair_body(p, prev):
        g0 = 2 * p
        g1 = g0 + 1
        start(g1, xbuf1, ibuf1, xsem1, isem1)
        wait(g0, xbuf0, ibuf0, xsem0, isem0)
        prev = process(xbuf0, ibuf0, prev)

        @pl.when(g0 + 2 < NBLK)
        def _():
            start(g0 + 2, xbuf0, ibuf0, xsem0, isem0)

        wait(g1, xbuf1, ibuf1, xsem1, isem1)
        prev = process(xbuf1, ibuf1, prev)
        return prev

    prev = lax.fori_loop(0, NBLK // 2, pair_body, jnp.int32(-1))

    sum_out[prev] = sacc[...]
    max_out[prev] = macc[...]

    pltpu.sync_copy(sum_out, out_hbm.at[:, pl.ds(c0, L)])
    pltpu.sync_copy(max_out, out_hbm.at[:, pl.ds(D + c0, L)])


@jax.jit
def _readout(x, ids):
    mesh = plsc.VectorSubcoreMesh(core_axis_name="c", subcore_axis_name="s")
    fn = pl.kernel(
        _sc_body,
        out_type=jax.ShapeDtypeStruct((NSEG, 2 * D), jnp.float32),
        mesh=mesh,
        compiler_params=pltpu.CompilerParams(use_tc_tiling_on_sc=False),
        scratch_types=[
            pltpu.VMEM((B, L), jnp.float32),
            pltpu.VMEM((B, L), jnp.float32),
            pltpu.VMEM((B,), jnp.int32),
            pltpu.VMEM((B,), jnp.int32),
            pltpu.VMEM((NSEG, L), jnp.float32),
            pltpu.VMEM((NSEG, L), jnp.float32),
            pltpu.VMEM((L,), jnp.float32),
            pltpu.VMEM((L,), jnp.float32),
            pltpu.SemaphoreType.DMA,
            pltpu.SemaphoreType.DMA,
            pltpu.SemaphoreType.DMA,
            pltpu.SemaphoreType.DMA,
        ],
    )
    return fn(x, ids)


def kernel(x, batch):
    return _readout(x, batch.astype(jnp.int32))


# speedup vs baseline: 4.2073x; 1.0150x over previous
import functools

import jax
import jax.numpy as jnp
from jax import lax
from jax.experimental import pallas as pl
from jax.experimental.pallas import tpu as pltpu
from jax.experimental.pallas import tpu_sc as plsc

NROWS = 100000
D = 512
NSEG = 512
L = 16
NC = 2
NS = 16
B = 2000
NBLK = NROWS // B


def _sc_body(x_hbm, ids_hbm, out_hbm,
             xbuf0, xbuf1, ibuf0, ibuf1, sum_out, max_out, sacc, macc,
             xsem0, xsem1, isem0, isem1):
    c = lax.axis_index("c")
    s = lax.axis_index("s")
    wid = s * NC + c
    c0 = wid * L

    zeros = jnp.zeros((L,), jnp.float32)
    ninf = jnp.full((L,), -jnp.inf, jnp.float32)

    def init_body(i, carry):
        sum_out[i] = zeros
        max_out[i] = ninf
        return carry
    lax.fori_loop(0, NSEG, init_body, 0)

    def start(g, xb, ib, xsem, isem):
        pltpu.async_copy(x_hbm.at[pl.ds(g * B, B), pl.ds(c0, L)], xb, xsem)
        pltpu.async_copy(ids_hbm.at[pl.ds(g * B, B)], ib, isem)

    def wait(g, xb, ib, xsem, isem):
        pltpu.make_async_copy(
            x_hbm.at[pl.ds(g * B, B), pl.ds(c0, L)], xb, xsem).wait()
        pltpu.make_async_copy(
            ids_hbm.at[pl.ds(g * B, B)], ib, isem).wait()

    def _tree(op, xs):
        while len(xs) > 1:
            xs = [op(xs[i], xs[i + 1]) for i in range(0, len(xs), 2)]
        return xs[0]

    def group(base, prev, sa, ma, ib, xb):
        idvec = ib[pl.ds(base, L)]
        vs = [xb[base + j] for j in range(L)]
        same = jnp.logical_and(idvec[0] == idvec[L - 1],
                               idvec[0] == prev)
        S = _tree(lambda a, b: a + b, vs)
        M = _tree(jnp.maximum, vs)

        @pl.when(jnp.logical_not(same))
        def slow():
            p = prev
            sa_s = sa
            ma_s = ma
            for j in range(L):
                sid = idvec[j]
                new = sid != p

                @pl.when(jnp.logical_and(new, p >= 0))
                def _(sa_s=sa_s, ma_s=ma_s, p=p):
                    sacc[...] = sa_s
                    macc[...] = ma_s
                    sum_out[p] = sacc[...]
                    max_out[p] = macc[...]

                sa_s = jnp.where(new, vs[j], sa_s + vs[j])
                ma_s = jnp.where(new, vs[j], jnp.maximum(ma_s, vs[j]))
                p = sid
            sacc[...] = sa_s
            macc[...] = ma_s

        sa = jnp.where(same, sa + S, sacc[...])
        ma = jnp.where(same, jnp.maximum(ma, M), macc[...])
        return idvec[L - 1], sa, ma

    def process(xb, ib, carry):
        def group_body(k, carry):
            prev, sa, ma = carry
            prev, sa, ma = group(k * (2 * L), prev, sa, ma, ib, xb)
            prev, sa, ma = group(k * (2 * L) + L, prev, sa, ma, ib, xb)
            return prev, sa, ma
        carry = lax.fori_loop(0, B // (2 * L), group_body, carry)
        if B % (2 * L) != 0:
            assert B % (2 * L) == L
            prev, sa, ma = carry
            carry = group(B - L, prev, sa, ma, ib, xb)
        return carry

    start(0, xbuf0, ibuf0, xsem0, isem0)
    sacc[...] = zeros
    macc[...] = ninf

    def pair_body(p, carry):
        g0 = 2 * p
        g1 = g0 + 1
        start(g1, xbuf1, ibuf1, xsem1, isem1)
        wait(g0, xbuf0, ibuf0, xsem0, isem0)
        carry = process(xbuf0, ibuf0, carry)

        @pl.when(g0 + 2 < NBLK)
        def _():
            start(g0 + 2, xbuf0, ibuf0, xsem0, isem0)

        wait(g1, xbuf1, ibuf1, xsem1, isem1)
        carry = process(xbuf1, ibuf1, carry)
        return carry

    prev, sa, ma = lax.fori_loop(0, NBLK // 2, pair_body,
                                 (jnp.int32(-1), zeros, ninf))

    sacc[...] = sa
    macc[...] = ma
    sum_out[prev] = sacc[...]
    max_out[prev] = macc[...]

    pltpu.sync_copy(sum_out, out_hbm.at[:, pl.ds(c0, L)])
    pltpu.sync_copy(max_out, out_hbm.at[:, pl.ds(D + c0, L)])


@jax.jit
def _readout(x, ids):
    mesh = plsc.VectorSubcoreMesh(core_axis_name="c", subcore_axis_name="s")
    fn = pl.kernel(
        _sc_body,
        out_type=jax.ShapeDtypeStruct((NSEG, 2 * D), jnp.float32),
        mesh=mesh,
        compiler_params=pltpu.CompilerParams(use_tc_tiling_on_sc=False),
        scratch_types=[
            pltpu.VMEM((B, L), jnp.float32),
            pltpu.VMEM((B, L), jnp.float32),
            pltpu.VMEM((B,), jnp.int32),
            pltpu.VMEM((B,), jnp.int32),
            pltpu.VMEM((NSEG, L), jnp.float32),
            pltpu.VMEM((NSEG, L), jnp.float32),
            pltpu.VMEM((L,), jnp.float32),
            pltpu.VMEM((L,), jnp.float32),
            pltpu.SemaphoreType.DMA,
            pltpu.SemaphoreType.DMA,
            pltpu.SemaphoreType.DMA,
            pltpu.SemaphoreType.DMA,
        ],
    )
    return fn(x, ids)


def kernel(x, batch):
    return _readout(x, batch.astype(jnp.int32))
